# Initial kernel scaffold; baseline (speedup 1.0000x reference)
import jax
import jax.numpy as jnp
from jax.experimental import pallas as pl


def kernel(x, edge_index, params):
    raise NotImplementedError("write your pallas kernel here")


# speedup vs baseline: 35.1545x; 35.1545x over previous
import jax
import jax.numpy as jnp
from jax import lax
from jax.experimental import pallas as pl
from jax.experimental.pallas import tpu as pltpu
from jax.experimental.pallas import tpu_sc as plsc

_N = 10000
_E = 320000
_D = 128
_HID = 256
_NP = 10240
_NC = 2
_NS = 16
_NW = _NC * _NS
_EPW = _E // _NW
_L = 16
_G = 128
_NJ = 16
_NA = _NP
_RPT = _NA // _NS
_BR = 2000
_HALF = _NP // _NC
_NAH = _HALF + 512
_RPTH = _NAH // _NS
_EPT = _E // _NS
_EB = 2000
_NBLK = (_EPT + 2 * _G - 1) // _G

_mesh = plsc.VectorSubcoreMesh(core_axis_name="c", subcore_axis_name="s")


def _sc_deg_mask1_body(src_hbm, dst_hbm, deg_out, m1_out,
                       src_v, dst_v, deg_v, m1_v):
    wid = lax.axis_index("s") * _NC + lax.axis_index("c")
    base = wid * _EPW
    pltpu.sync_copy(src_hbm.at[pl.ds(base, _EPW)], src_v)
    pltpu.sync_copy(dst_hbm.at[pl.ds(base, _EPW)], dst_v)
    zi = jnp.zeros((_L,), jnp.int32)
    zf = jnp.zeros((_L,), jnp.float32)

    def zero_body(i, c):
        deg_v[pl.ds(i * _L, _L)] = zf
        m1_v[pl.ds(i * _L, _L)] = zi
        return c
    lax.fori_loop(0, _NP // _L, zero_body, 0)

    ones = jnp.ones((_L,), jnp.int32)
    onesf = jnp.ones((_L,), jnp.float32)

    def body(i, c):
        s = src_v[pl.ds(i * _L, _L)]
        t = dst_v[pl.ds(i * _L, _L)]
        plsc.addupdate_scatter(deg_v, [t], onesf)
        plsc.store_scatter(m1_v, [s], ones, mask=t == 0)
        return c
    lax.fori_loop(0, _EPW // _L, body, 0)
    pltpu.sync_copy(deg_v, deg_out.at[wid])
    pltpu.sync_copy(m1_v, m1_out.at[wid])


_sc_deg_mask1 = pl.kernel(
    _sc_deg_mask1_body,
    out_type=(jax.ShapeDtypeStruct((_NW, _NP), jnp.float32),
              jax.ShapeDtypeStruct((_NW, _NP), jnp.int32)),
    mesh=_mesh,
    compiler_params=pltpu.CompilerParams(needs_layout_passes=False),
    scratch_types=(pltpu.VMEM((_EPW,), jnp.int32),
                   pltpu.VMEM((_EPW,), jnp.int32),
                   pltpu.VMEM((_NP,), jnp.float32),
                   pltpu.VMEM((_NP,), jnp.int32)),
)


def _sc_mask2_body(src_hbm, dst_hbm, m1_hbm, m2_out,
                   src_v, dst_v, m1_v, m2_v):
    wid = lax.axis_index("s") * _NC + lax.axis_index("c")
    base = wid * _EPW
    pltpu.sync_copy(src_hbm.at[pl.ds(base, _EPW)], src_v)
    pltpu.sync_copy(dst_hbm.at[pl.ds(base, _EPW)], dst_v)
    pltpu.sync_copy(m1_hbm, m1_v)
    zi = jnp.zeros((_L,), jnp.int32)

    def zero_body(i, c):
        m2_v[pl.ds(i * _L, _L)] = zi
        return c
    lax.fori_loop(0, _NP // _L, zero_body, 0)

    ones = jnp.ones((_L,), jnp.int32)

    def body(i, c):
        s = src_v[pl.ds(i * _L, _L)]
        t = dst_v[pl.ds(i * _L, _L)]
        mv = plsc.load_gather(m1_v, [t])
        plsc.store_scatter(m2_v, [s], ones, mask=mv > 0)
        return c
    lax.fori_loop(0, _EPW // _L, body, 0)
    pltpu.sync_copy(m2_v, m2_out.at[wid])


_sc_mask2 = pl.kernel(
    _sc_mask2_body,
    out_type=jax.ShapeDtypeStruct((_NW, _NP), jnp.int32),
    mesh=_mesh,
    compiler_params=pltpu.CompilerParams(needs_layout_passes=False),
    scratch_types=(pltpu.VMEM((_EPW,), jnp.int32),
                   pltpu.VMEM((_EPW,), jnp.int32),
                   pltpu.VMEM((_NP,), jnp.int32),
                   pltpu.VMEM((_NP,), jnp.int32)),
)


def _sc_scatter_body(src_hbm, dst_hbm, mask_hbm, rows_hbm, acc_out,
                     src_v, dst_v, mask_v, csrc_v, cdst_v,
                     rowbuf, acc_sh, sem):
    cid = lax.axis_index("c")
    sid = lax.axis_index("s")
    wid = sid * _NC + cid
    base = sid * _EPT
    pltpu.sync_copy(mask_hbm, mask_v)

    zf = jnp.zeros((_L,), jnp.float32)

    def zrow_body(i, c):
        for k in range(_D // _L):
            rowbuf[i, pl.ds(k * _L, _L)] = zf
        return c
    lax.fori_loop(0, _G, zrow_body, 0)

    rbase = sid * _RPTH
    nfull = _RPTH // _G
    rem = _RPTH - nfull * _G
    for k in range(nfull):
        pltpu.sync_copy(rowbuf, acc_sh.at[pl.ds(rbase + k * _G, _G)])
    pltpu.sync_copy(rowbuf.at[pl.ds(0, rem)],
                    acc_sh.at[pl.ds(rbase + nfull * _G, rem)])
    plsc.subcore_barrier()

    spread = jnp.zeros((_L,), jnp.int32) + (wid % _NJ)
    junk = jnp.zeros((_L,), jnp.int32) + (_HALF + (wid % _NJ))

    def pf_body(i, c):
        for k in range(_G // _L):
            csrc_v[i, pl.ds(k * _L, _L)] = spread
            cdst_v[i, pl.ds(k * _L, _L)] = junk
        return c
    lax.fori_loop(0, _NBLK, pf_body, 0)

    lo = cid * _HALF

    def ob_body(ob, off):
        pltpu.sync_copy(src_hbm.at[pl.ds(base + ob * _EB, _EB)], src_v)
        pltpu.sync_copy(dst_hbm.at[pl.ds(base + ob * _EB, _EB)], dst_v)

        def cb(i, off):
            s = src_v[pl.ds(i * _L, _L)]
            t = dst_v[pl.ds(i * _L, _L)]
            tl = t - lo
            mv = plsc.load_gather(mask_v, [t])
            m = (mv > 0) & (tl >= 0) & (tl < _HALF)
            mi = jnp.where(m, 1, 0)
            pos = jnp.cumsum(mi) + (off - 1)
            prow = lax.shift_right_logical(pos, 7)
            pcol = lax.bitwise_and(pos, _G - 1)
            plsc.store_scatter(csrc_v, [prow, pcol], s, mask=m)
            plsc.store_scatter(cdst_v, [prow, pcol], tl, mask=m)
            return off + jnp.sum(mi)
        return lax.fori_loop(0, _EB // _L, cb, off)
    kact = lax.fori_loop(0, _EPT // _EB, ob_body, jnp.int32(0))

    nb = (kact + _G - 1) // _G

    def fb(j, c):
        pltpu.async_copy(rows_hbm.at[csrc_v.at[j]], rowbuf, sem).wait()
        pltpu.sync_copy(rowbuf, acc_sh.at[cdst_v.at[j]], add=True)
        return c
    lax.fori_loop(0, nb, fb, 0)
    plsc.subcore_barrier()

    for k in range(nfull):
        pltpu.sync_copy(acc_sh.at[pl.ds(rbase + k * _G, _G)], rowbuf)
        pltpu.sync_copy(rowbuf, acc_out.at[cid, pl.ds(rbase + k * _G, _G)])
    pltpu.sync_copy(acc_sh.at[pl.ds(rbase + nfull * _G, rem)],
                    rowbuf.at[pl.ds(0, rem)])
    pltpu.sync_copy(rowbuf.at[pl.ds(0, rem)],
                    acc_out.at[cid, pl.ds(rbase + nfull * _G, rem)])


_sc_scatter = pl.kernel(
    _sc_scatter_body,
    out_type=jax.ShapeDtypeStruct((_NC, _NAH, _D), jnp.float32),
    mesh=_mesh,
    compiler_params=pltpu.CompilerParams(needs_layout_passes=False),
    scratch_types=(pltpu.VMEM((_EB,), jnp.int32),
                   pltpu.VMEM((_EB,), jnp.int32),
                   pltpu.VMEM((_NP,), jnp.int32),
                   pltpu.VMEM((_NBLK, _G), jnp.int32),
                   pltpu.VMEM((_NBLK, _G), jnp.int32),
                   pltpu.VMEM((_G, _D), jnp.float32),
                   pltpu.VMEM_SHARED((_NAH, _D), jnp.float32),
                   pltpu.SemaphoreType.DMA),
)


def _tc_reduce1_body(degp_ref, m1p_ref, dinv_ref, m1_ref):
    degs = jnp.sum(degp_ref[...], axis=0, keepdims=True) + 1.0
    dinv_ref[...] = lax.rsqrt(degs)
    m1 = jnp.sum(m1p_ref[...], axis=0, keepdims=True) > 0
    col = lax.broadcasted_iota(jnp.int32, (1, _NP), 1)
    m1_ref[...] = jnp.where((col == 0) | m1, 1, 0).astype(jnp.int32)


_tc_reduce1 = pl.pallas_call(
    _tc_reduce1_body,
    out_shape=(jax.ShapeDtypeStruct((1, _NP), jnp.float32),
               jax.ShapeDtypeStruct((1, _NP), jnp.int32)),
)


def _tc_mask2_body(m2p_ref, m1_ref, m2_ref):
    m2 = jnp.sum(m2p_ref[...], axis=0, keepdims=True) > 0
    m2_ref[...] = jnp.where(m2 | (m1_ref[...] > 0), 1, 0).astype(jnp.int32)


_tc_mask2 = pl.pallas_call(
    _tc_mask2_body,
    out_shape=jax.ShapeDtypeStruct((1, _NP), jnp.int32),
)


def _tc_scalemm_body(h_ref, w_ref, dinv_ref, out_ref):
    out_ref[...] = dinv_ref[...] * jnp.dot(
        h_ref[...], w_ref[...], preferred_element_type=jnp.float32)


_tc_scalemm = pl.pallas_call(
    _tc_scalemm_body,
    grid=(_N // _BR,),
    in_specs=[pl.BlockSpec((_BR, _D), lambda i: (i, 0)),
              pl.BlockSpec((_D, _D), lambda i: (0, 0)),
              pl.BlockSpec((_BR, 1), lambda i: (i, 0))],
    out_specs=pl.BlockSpec((_BR, _D), lambda i: (i, 0)),
    out_shape=jax.ShapeDtypeStruct((_N, _D), jnp.float32),
)


def _tc_post_body(acc_ref, hs_ref, dinv_ref, b_ref, g_ref,
                  bn_ref, hprev_ref, out_ref):
    pre = dinv_ref[...] * (acc_ref[...] + hs_ref[...]) + b_ref[...]
    mu = jnp.mean(pre, axis=1, keepdims=True)
    var = jnp.mean((pre - mu) ** 2, axis=1, keepdims=True)
    ln = (pre - mu) * lax.rsqrt(var + 1e-5) * g_ref[...] + bn_ref[...]
    out_ref[...] = jnp.maximum(ln, 0.0) + hprev_ref[...]


_tc_post = pl.pallas_call(
    _tc_post_body,
    grid=(_N // _BR,),
    in_specs=[pl.BlockSpec((_BR, _D), lambda i: (i, 0)),
              pl.BlockSpec((_BR, _D), lambda i: (i, 0)),
              pl.BlockSpec((_BR, 1), lambda i: (i, 0)),
              pl.BlockSpec((1, _D), lambda i: (0, 0)),
              pl.BlockSpec((1, _D), lambda i: (0, 0)),
              pl.BlockSpec((1, _D), lambda i: (0, 0)),
              pl.BlockSpec((_BR, _D), lambda i: (i, 0))],
    out_specs=pl.BlockSpec((_BR, _D), lambda i: (i, 0)),
    out_shape=jax.ShapeDtypeStruct((_N, _D), jnp.float32),
)


def _tc_heads_body(acc0, hs3r, h2r, dinv0, b3, g3, bn3,
                   cvw1, cvb1, cvw2, cvb2, caw1, cab1, caw2, cab2,
                   nvw1, nvb1, nvw2, nvb2, naw1, nab1, naw2, nab2,
                   clsq_ref, navq_ref):
    pre = dinv0[...] * (acc0[...] + hs3r[...]) + b3[...]
    mu = jnp.mean(pre, axis=1, keepdims=True)
    var = jnp.mean((pre - mu) ** 2, axis=1, keepdims=True)
    ln = (pre - mu) * lax.rsqrt(var + 1e-5) * g3[...] + bn3[...]
    cur = jnp.maximum(ln, 0.0) + h2r[...]

    def mlp(w1, b1, w2, b2):
        hmid = jnp.maximum(
            jnp.dot(cur, w1[...], preferred_element_type=jnp.float32)
            + b1[...], 0.0)
        return jnp.dot(hmid, w2[...],
                       preferred_element_type=jnp.float32) + b2[...]

    cv = mlp(cvw1, cvb1, cvw2, cvb2)
    ca = mlp(caw1, cab1, caw2, cab2)
    clsq_ref[...] = cv + ca - jnp.mean(ca, axis=1, keepdims=True)
    nv = mlp(nvw1, nvb1, nvw2, nvb2)
    na = mlp(naw1, nab1, naw2, nab2)
    navq_ref[...] = nv + na - jnp.mean(na, axis=1, keepdims=True)


_tc_heads = pl.pallas_call(
    _tc_heads_body,
    out_shape=(jax.ShapeDtypeStruct((1, 10), jnp.float32),
               jax.ShapeDtypeStruct((1, 32), jnp.float32)),
)


def _reasm(acc):
    return jnp.concatenate([acc[0, :_HALF], acc[1, :_HALF]], axis=0)[:_N]


def kernel(x, edge_index, params):
    p = params
    ei = edge_index.astype(jnp.int32)
    src = ei[0]
    dst = ei[1]

    degp, m1p = _sc_deg_mask1(src, dst)
    dinv2d, mask1_2d = _tc_reduce1(degp, m1p)
    mask1 = mask1_2d.reshape(_NP)
    m2p = _sc_mask2(src, dst, mask1)
    mask2 = _tc_mask2(m2p, mask1_2d).reshape(_NP)
    dinv_col = dinv2d[0, :_N].reshape(_N, 1)

    hs1 = _tc_scalemm(x, p['W1'], dinv_col)
    acc1 = _sc_scatter(src, dst, mask2, hs1)
    h1 = _tc_post(_reasm(acc1), hs1, dinv_col,
                  p['b1'].reshape(1, _D), p['ln1_g'].reshape(1, _D),
                  p['ln1_b'].reshape(1, _D), x)

    hs2 = _tc_scalemm(h1, p['W2'], dinv_col)
    acc2 = _sc_scatter(src, dst, mask1, hs2)
    h2 = _tc_post(_reasm(acc2), hs2, dinv_col,
                  p['b2'].reshape(1, _D), p['ln2_g'].reshape(1, _D),
                  p['ln2_b'].reshape(1, _D), h1)

    hs3 = _tc_scalemm(h2, p['W3'], dinv_col)
    mask0 = jnp.zeros((_NP,), jnp.int32).at[0].set(1)
    acc3 = _sc_scatter(src, dst, mask0, hs3)

    cls_q, nav_q = _tc_heads(
        acc3[0, :1, :], hs3[:1], h2[:1], dinv2d[:, :1],
        p['b3'].reshape(1, _D), p['ln3_g'].reshape(1, _D),
        p['ln3_b'].reshape(1, _D),
        p['cv_W1'], p['cv_b1'].reshape(1, _HID), p['cv_W2'],
        p['cv_b2'].reshape(1, 1),
        p['ca_W1'], p['ca_b1'].reshape(1, _HID), p['ca_W2'],
        p['ca_b2'].reshape(1, 10),
        p['nv_W1'], p['nv_b1'].reshape(1, _HID), p['nv_W2'],
        p['nv_b2'].reshape(1, 1),
        p['na_W1'], p['na_b1'].reshape(1, _HID), p['na_W2'],
        p['na_b2'].reshape(1, 32))
    return (cls_q, nav_q)


# speedup vs baseline: 35.2076x; 1.0015x over previous
import jax
import jax.numpy as jnp
from jax import lax
from jax.experimental import pallas as pl
from jax.experimental.pallas import tpu as pltpu
from jax.experimental.pallas import tpu_sc as plsc

_N = 10000
_E = 320000
_D = 128
_HID = 256
_NP = 10240
_NC = 2
_NS = 16
_NW = _NC * _NS
_EPW = _E // _NW
_L = 16
_G = 128
_NJ = 16
_NA = _NP
_RPT = _NA // _NS
_BR = 2000
_HALF = _NP // _NC
_NAH = _HALF + 512
_RPTH = _NAH // _NS
_EPT = _E // _NS
_EB = 2000
_NBLK = (_EPT + 2 * _G - 1) // _G

_mesh = plsc.VectorSubcoreMesh(core_axis_name="c", subcore_axis_name="s")


def _sc_deg_mask1_body(src_hbm, dst_hbm, deg_out, m1_out,
                       src_v, dst_v, deg_v, m1_v):
    wid = lax.axis_index("s") * _NC + lax.axis_index("c")
    base = wid * _EPW
    pltpu.sync_copy(src_hbm.at[pl.ds(base, _EPW)], src_v)
    pltpu.sync_copy(dst_hbm.at[pl.ds(base, _EPW)], dst_v)
    zi = jnp.zeros((_L,), jnp.int32)
    zf = jnp.zeros((_L,), jnp.float32)

    def zero_body(i, c):
        deg_v[pl.ds(i * _L, _L)] = zf
        m1_v[pl.ds(i * _L, _L)] = zi
        return c
    lax.fori_loop(0, _NP // _L, zero_body, 0)

    ones = jnp.ones((_L,), jnp.int32)
    onesf = jnp.ones((_L,), jnp.float32)

    def body(i, c):
        s = src_v[pl.ds(i * _L, _L)]
        t = dst_v[pl.ds(i * _L, _L)]
        plsc.addupdate_scatter(deg_v, [t], onesf)
        plsc.store_scatter(m1_v, [s], ones, mask=t == 0)
        return c
    lax.fori_loop(0, _EPW // _L, body, 0)
    pltpu.sync_copy(deg_v, deg_out.at[wid])
    pltpu.sync_copy(m1_v, m1_out.at[wid])


_sc_deg_mask1 = pl.kernel(
    _sc_deg_mask1_body,
    out_type=(jax.ShapeDtypeStruct((_NW, _NP), jnp.float32),
              jax.ShapeDtypeStruct((_NW, _NP), jnp.int32)),
    mesh=_mesh,
    compiler_params=pltpu.CompilerParams(needs_layout_passes=False),
    scratch_types=(pltpu.VMEM((_EPW,), jnp.int32),
                   pltpu.VMEM((_EPW,), jnp.int32),
                   pltpu.VMEM((_NP,), jnp.float32),
                   pltpu.VMEM((_NP,), jnp.int32)),
)


def _sc_mask2_body(src_hbm, dst_hbm, m1_hbm, m2_out,
                   src_v, dst_v, m1_v, m2_v):
    wid = lax.axis_index("s") * _NC + lax.axis_index("c")
    base = wid * _EPW
    pltpu.sync_copy(src_hbm.at[pl.ds(base, _EPW)], src_v)
    pltpu.sync_copy(dst_hbm.at[pl.ds(base, _EPW)], dst_v)
    pltpu.sync_copy(m1_hbm, m1_v)
    zi = jnp.zeros((_L,), jnp.int32)

    def zero_body(i, c):
        m2_v[pl.ds(i * _L, _L)] = zi
        return c
    lax.fori_loop(0, _NP // _L, zero_body, 0)

    ones = jnp.ones((_L,), jnp.int32)

    def body(i, c):
        s = src_v[pl.ds(i * _L, _L)]
        t = dst_v[pl.ds(i * _L, _L)]
        mv = plsc.load_gather(m1_v, [t])
        plsc.store_scatter(m2_v, [s], ones, mask=mv > 0)
        return c
    lax.fori_loop(0, _EPW // _L, body, 0)
    pltpu.sync_copy(m2_v, m2_out.at[wid])


_sc_mask2 = pl.kernel(
    _sc_mask2_body,
    out_type=jax.ShapeDtypeStruct((_NW, _NP), jnp.int32),
    mesh=_mesh,
    compiler_params=pltpu.CompilerParams(needs_layout_passes=False),
    scratch_types=(pltpu.VMEM((_EPW,), jnp.int32),
                   pltpu.VMEM((_EPW,), jnp.int32),
                   pltpu.VMEM((_NP,), jnp.int32),
                   pltpu.VMEM((_NP,), jnp.int32)),
)


def _sc_scatter_body(src_hbm, dst_hbm, mask_hbm, rows_hbm, acc_out,
                     src_v, dst_v, mask_v, csrc_v, cdst_v,
                     sstage, dstage, rowbuf, acc_sh, sem):
    cid = lax.axis_index("c")
    sid = lax.axis_index("s")
    wid = sid * _NC + cid
    base = sid * _EPT
    pltpu.sync_copy(mask_hbm, mask_v)

    zf = jnp.zeros((_L,), jnp.float32)

    def zrow_body(i, c):
        for k in range(_D // _L):
            rowbuf[i, pl.ds(k * _L, _L)] = zf
        return c
    lax.fori_loop(0, _G, zrow_body, 0)

    rbase = sid * _RPTH
    nfull = _RPTH // _G
    rem = _RPTH - nfull * _G
    for k in range(nfull):
        pltpu.sync_copy(rowbuf, acc_sh.at[pl.ds(rbase + k * _G, _G)])
    pltpu.sync_copy(rowbuf.at[pl.ds(0, rem)],
                    acc_sh.at[pl.ds(rbase + nfull * _G, rem)])
    plsc.subcore_barrier()

    spread = jnp.zeros((_L,), jnp.int32) + (wid % _NJ)
    junk = jnp.zeros((_L,), jnp.int32) + (_HALF + (wid % _NJ))

    def pf_body(i, c):
        csrc_v[pl.ds(i * _L, _L)] = spread
        cdst_v[pl.ds(i * _L, _L)] = junk
        return c
    lax.fori_loop(0, _NBLK * _G // _L, pf_body, 0)

    lo = cid * _HALF

    def ob_body(ob, off):
        pltpu.sync_copy(src_hbm.at[pl.ds(base + ob * _EB, _EB)], src_v)
        pltpu.sync_copy(dst_hbm.at[pl.ds(base + ob * _EB, _EB)], dst_v)

        def cb(i, off):
            s = src_v[pl.ds(i * _L, _L)]
            t = dst_v[pl.ds(i * _L, _L)]
            tl = t - lo
            mv = plsc.load_gather(mask_v, [t])
            m = (mv > 0) & (tl >= 0) & (tl < _HALF)
            plsc.store_compressed(csrc_v.at[pl.ds(off, _L)], s, mask=m)
            plsc.store_compressed(cdst_v.at[pl.ds(off, _L)], tl, mask=m)
            cnt = plsc.all_reduce_population_count(m)
            return off + cnt[0]
        return lax.fori_loop(0, _EB // _L, cb, off)
    kact = lax.fori_loop(0, _EPT // _EB, ob_body, jnp.int32(0))

    nb = (kact + _G - 1) // _G

    def fb(j, c):
        for k in range(_G // _L):
            sstage[pl.ds(k * _L, _L)] = csrc_v[pl.ds(j * _G + k * _L, _L)]
            dstage[pl.ds(k * _L, _L)] = cdst_v[pl.ds(j * _G + k * _L, _L)]
        pltpu.async_copy(rows_hbm.at[sstage], rowbuf, sem).wait()
        pltpu.sync_copy(rowbuf, acc_sh.at[dstage], add=True)
        return c
    lax.fori_loop(0, nb, fb, 0)
    plsc.subcore_barrier()

    for k in range(nfull):
        pltpu.sync_copy(acc_sh.at[pl.ds(rbase + k * _G, _G)], rowbuf)
        pltpu.sync_copy(rowbuf, acc_out.at[cid, pl.ds(rbase + k * _G, _G)])
    pltpu.sync_copy(acc_sh.at[pl.ds(rbase + nfull * _G, rem)],
                    rowbuf.at[pl.ds(0, rem)])
    pltpu.sync_copy(rowbuf.at[pl.ds(0, rem)],
                    acc_out.at[cid, pl.ds(rbase + nfull * _G, rem)])


_sc_scatter = pl.kernel(
    _sc_scatter_body,
    out_type=jax.ShapeDtypeStruct((_NC, _NAH, _D), jnp.float32),
    mesh=_mesh,
    compiler_params=pltpu.CompilerParams(needs_layout_passes=False),
    scratch_types=(pltpu.VMEM((_EB,), jnp.int32),
                   pltpu.VMEM((_EB,), jnp.int32),
                   pltpu.VMEM((_NP,), jnp.int32),
                   pltpu.VMEM((_NBLK * _G,), jnp.int32),
                   pltpu.VMEM((_NBLK * _G,), jnp.int32),
                   pltpu.VMEM((_G,), jnp.int32),
                   pltpu.VMEM((_G,), jnp.int32),
                   pltpu.VMEM((_G, _D), jnp.float32),
                   pltpu.VMEM_SHARED((_NAH, _D), jnp.float32),
                   pltpu.SemaphoreType.DMA),
)


def _tc_reduce1_body(degp_ref, m1p_ref, dinv_ref, m1_ref):
    degs = jnp.sum(degp_ref[...], axis=0, keepdims=True) + 1.0
    dinv_ref[...] = lax.rsqrt(degs)
    m1 = jnp.sum(m1p_ref[...], axis=0, keepdims=True) > 0
    col = lax.broadcasted_iota(jnp.int32, (1, _NP), 1)
    m1_ref[...] = jnp.where((col == 0) | m1, 1, 0).astype(jnp.int32)


_tc_reduce1 = pl.pallas_call(
    _tc_reduce1_body,
    out_shape=(jax.ShapeDtypeStruct((1, _NP), jnp.float32),
               jax.ShapeDtypeStruct((1, _NP), jnp.int32)),
)


def _tc_mask2_body(m2p_ref, m1_ref, m2_ref):
    m2 = jnp.sum(m2p_ref[...], axis=0, keepdims=True) > 0
    m2_ref[...] = jnp.where(m2 | (m1_ref[...] > 0), 1, 0).astype(jnp.int32)


_tc_mask2 = pl.pallas_call(
    _tc_mask2_body,
    out_shape=jax.ShapeDtypeStruct((1, _NP), jnp.int32),
)


def _tc_scalemm_body(h_ref, w_ref, dinv_ref, out_ref):
    out_ref[...] = dinv_ref[...] * jnp.dot(
        h_ref[...], w_ref[...], preferred_element_type=jnp.float32)


_tc_scalemm = pl.pallas_call(
    _tc_scalemm_body,
    grid=(_N // _BR,),
    in_specs=[pl.BlockSpec((_BR, _D), lambda i: (i, 0)),
              pl.BlockSpec((_D, _D), lambda i: (0, 0)),
              pl.BlockSpec((_BR, 1), lambda i: (i, 0))],
    out_specs=pl.BlockSpec((_BR, _D), lambda i: (i, 0)),
    out_shape=jax.ShapeDtypeStruct((_N, _D), jnp.float32),
)


def _tc_post_body(acc_ref, hs_ref, dinv_ref, b_ref, g_ref,
                  bn_ref, hprev_ref, out_ref):
    pre = dinv_ref[...] * (acc_ref[...] + hs_ref[...]) + b_ref[...]
    mu = jnp.mean(pre, axis=1, keepdims=True)
    var = jnp.mean((pre - mu) ** 2, axis=1, keepdims=True)
    ln = (pre - mu) * lax.rsqrt(var + 1e-5) * g_ref[...] + bn_ref[...]
    out_ref[...] = jnp.maximum(ln, 0.0) + hprev_ref[...]


_tc_post = pl.pallas_call(
    _tc_post_body,
    grid=(_N // _BR,),
    in_specs=[pl.BlockSpec((_BR, _D), lambda i: (i, 0)),
              pl.BlockSpec((_BR, _D), lambda i: (i, 0)),
              pl.BlockSpec((_BR, 1), lambda i: (i, 0)),
              pl.BlockSpec((1, _D), lambda i: (0, 0)),
              pl.BlockSpec((1, _D), lambda i: (0, 0)),
              pl.BlockSpec((1, _D), lambda i: (0, 0)),
              pl.BlockSpec((_BR, _D), lambda i: (i, 0))],
    out_specs=pl.BlockSpec((_BR, _D), lambda i: (i, 0)),
    out_shape=jax.ShapeDtypeStruct((_N, _D), jnp.float32),
)


def _tc_heads_body(acc0, hs3r, h2r, dinv0, b3, g3, bn3,
                   cvw1, cvb1, cvw2, cvb2, caw1, cab1, caw2, cab2,
                   nvw1, nvb1, nvw2, nvb2, naw1, nab1, naw2, nab2,
                   clsq_ref, navq_ref):
    pre = dinv0[...] * (acc0[...] + hs3r[...]) + b3[...]
    mu = jnp.mean(pre, axis=1, keepdims=True)
    var = jnp.mean((pre - mu) ** 2, axis=1, keepdims=True)
    ln = (pre - mu) * lax.rsqrt(var + 1e-5) * g3[...] + bn3[...]
    cur = jnp.maximum(ln, 0.0) + h2r[...]

    def mlp(w1, b1, w2, b2):
        hmid = jnp.maximum(
            jnp.dot(cur, w1[...], preferred_element_type=jnp.float32)
            + b1[...], 0.0)
        return jnp.dot(hmid, w2[...],
                       preferred_element_type=jnp.float32) + b2[...]

    cv = mlp(cvw1, cvb1, cvw2, cvb2)
    ca = mlp(caw1, cab1, caw2, cab2)
    clsq_ref[...] = cv + ca - jnp.mean(ca, axis=1, keepdims=True)
    nv = mlp(nvw1, nvb1, nvw2, nvb2)
    na = mlp(naw1, nab1, naw2, nab2)
    navq_ref[...] = nv + na - jnp.mean(na, axis=1, keepdims=True)


_tc_heads = pl.pallas_call(
    _tc_heads_body,
    out_shape=(jax.ShapeDtypeStruct((1, 10), jnp.float32),
               jax.ShapeDtypeStruct((1, 32), jnp.float32)),
)


def _reasm(acc):
    return jnp.concatenate([acc[0, :_HALF], acc[1, :_HALF]], axis=0)[:_N]


def kernel(x, edge_index, params):
    p = params
    ei = edge_index.astype(jnp.int32)
    src = ei[0]
    dst = ei[1]

    degp, m1p = _sc_deg_mask1(src, dst)
    dinv2d, mask1_2d = _tc_reduce1(degp, m1p)
    mask1 = mask1_2d.reshape(_NP)
    m2p = _sc_mask2(src, dst, mask1)
    mask2 = _tc_mask2(m2p, mask1_2d).reshape(_NP)
    dinv_col = dinv2d[0, :_N].reshape(_N, 1)

    hs1 = _tc_scalemm(x, p['W1'], dinv_col)
    acc1 = _sc_scatter(src, dst, mask2, hs1)
    h1 = _tc_post(_reasm(acc1), hs1, dinv_col,
                  p['b1'].reshape(1, _D), p['ln1_g'].reshape(1, _D),
                  p['ln1_b'].reshape(1, _D), x)

    hs2 = _tc_scalemm(h1, p['W2'], dinv_col)
    acc2 = _sc_scatter(src, dst, mask1, hs2)
    h2 = _tc_post(_reasm(acc2), hs2, dinv_col,
                  p['b2'].reshape(1, _D), p['ln2_g'].reshape(1, _D),
                  p['ln2_b'].reshape(1, _D), h1)

    hs3 = _tc_scalemm(h2, p['W3'], dinv_col)
    mask0 = jnp.zeros((_NP,), jnp.int32).at[0].set(1)
    acc3 = _sc_scatter(src, dst, mask0, hs3)

    cls_q, nav_q = _tc_heads(
        acc3[0, :1, :], hs3[:1], h2[:1], dinv2d[:, :1],
        p['b3'].reshape(1, _D), p['ln3_g'].reshape(1, _D),
        p['ln3_b'].reshape(1, _D),
        p['cv_W1'], p['cv_b1'].reshape(1, _HID), p['cv_W2'],
        p['cv_b2'].reshape(1, 1),
        p['ca_W1'], p['ca_b1'].reshape(1, _HID), p['ca_W2'],
        p['ca_b2'].reshape(1, 10),
        p['nv_W1'], p['nv_b1'].reshape(1, _HID), p['nv_W2'],
        p['nv_b2'].reshape(1, 1),
        p['na_W1'], p['na_b1'].reshape(1, _HID), p['na_W2'],
        p['na_b2'].reshape(1, 32))
    return (cls_q, nav_q)


# speedup vs baseline: 37.7914x; 1.0734x over previous
import jax
import jax.numpy as jnp
from jax import lax
from jax.experimental import pallas as pl
from jax.experimental.pallas import tpu as pltpu
from jax.experimental.pallas import tpu_sc as plsc

_N = 10000
_E = 320000
_D = 128
_HID = 256
_NP = 10240
_NC = 2
_NS = 16
_NW = _NC * _NS
_EPW = _E // _NW
_L = 16
_G = 128
_NJ = 16
_NA = _NP
_RPT = _NA // _NS
_BR = 2000
_HALF = _NP // _NC
_NAH = _HALF + 512
_RPTH = _NAH // _NS
_EPT = _E // _NS
_EB = 2000
_NBLK = (_EPT + 2 * _G - 1) // _G
_CAP = _NBLK * _G
_GF = 64

_mesh = plsc.VectorSubcoreMesh(core_axis_name="c", subcore_axis_name="s")


def _sc_deg_mask1_body(src_hbm, dst_hbm, deg_out, m1_out,
                       src_v, dst_v, deg_v, m1_v):
    wid = lax.axis_index("s") * _NC + lax.axis_index("c")
    base = wid * _EPW
    pltpu.sync_copy(src_hbm.at[pl.ds(base, _EPW)], src_v)
    pltpu.sync_copy(dst_hbm.at[pl.ds(base, _EPW)], dst_v)
    zi = jnp.zeros((_L,), jnp.int32)
    zf = jnp.zeros((_L,), jnp.float32)

    def zero_body(i, c):
        deg_v[pl.ds(i * _L, _L)] = zf
        m1_v[pl.ds(i * _L, _L)] = zi
        return c
    lax.fori_loop(0, _NP // _L, zero_body, 0)

    ones = jnp.ones((_L,), jnp.int32)
    onesf = jnp.ones((_L,), jnp.float32)

    def body(i, c):
        s = src_v[pl.ds(i * _L, _L)]
        t = dst_v[pl.ds(i * _L, _L)]
        plsc.addupdate_scatter(deg_v, [t], onesf)
        plsc.store_scatter(m1_v, [s], ones, mask=t == 0)
        return c
    lax.fori_loop(0, _EPW // _L, body, 0)
    pltpu.sync_copy(deg_v, deg_out.at[wid])
    pltpu.sync_copy(m1_v, m1_out.at[wid])


_sc_deg_mask1 = pl.kernel(
    _sc_deg_mask1_body,
    out_type=(jax.ShapeDtypeStruct((_NW, _NP), jnp.float32),
              jax.ShapeDtypeStruct((_NW, _NP), jnp.int32)),
    mesh=_mesh,
    compiler_params=pltpu.CompilerParams(needs_layout_passes=False),
    scratch_types=(pltpu.VMEM((_EPW,), jnp.int32),
                   pltpu.VMEM((_EPW,), jnp.int32),
                   pltpu.VMEM((_NP,), jnp.float32),
                   pltpu.VMEM((_NP,), jnp.int32)),
)


def _sc_mask2_body(src_hbm, dst_hbm, m1_hbm, m2_out,
                   src_v, dst_v, m1_v, m2_v):
    wid = lax.axis_index("s") * _NC + lax.axis_index("c")
    base = wid * _EPW
    pltpu.sync_copy(src_hbm.at[pl.ds(base, _EPW)], src_v)
    pltpu.sync_copy(dst_hbm.at[pl.ds(base, _EPW)], dst_v)
    pltpu.sync_copy(m1_hbm, m1_v)
    zi = jnp.zeros((_L,), jnp.int32)

    def zero_body(i, c):
        m2_v[pl.ds(i * _L, _L)] = zi
        return c
    lax.fori_loop(0, _NP // _L, zero_body, 0)

    ones = jnp.ones((_L,), jnp.int32)

    def body(i, c):
        s = src_v[pl.ds(i * _L, _L)]
        t = dst_v[pl.ds(i * _L, _L)]
        mv = plsc.load_gather(m1_v, [t])
        plsc.store_scatter(m2_v, [s], ones, mask=mv > 0)
        return c
    lax.fori_loop(0, _EPW // _L, body, 0)
    pltpu.sync_copy(m2_v, m2_out.at[wid])


_sc_mask2 = pl.kernel(
    _sc_mask2_body,
    out_type=jax.ShapeDtypeStruct((_NW, _NP), jnp.int32),
    mesh=_mesh,
    compiler_params=pltpu.CompilerParams(needs_layout_passes=False),
    scratch_types=(pltpu.VMEM((_EPW,), jnp.int32),
                   pltpu.VMEM((_EPW,), jnp.int32),
                   pltpu.VMEM((_NP,), jnp.int32),
                   pltpu.VMEM((_NP,), jnp.int32)),
)


def _sc_scatter_body(src_hbm, dst_hbm, mask_hbm, rows_hbm, acc_out,
                     src_v, dst_v, mask_v, csrc_v, cdst_v,
                     ss0, sd0, ss1, sd1, rb0, rb1,
                     msk_sh, acc_sh, sem0, sem1):
    cid = lax.axis_index("c")
    sid = lax.axis_index("s")
    wid = sid * _NC + cid
    base = sid * _EPT

    @pl.when(sid == 0)
    def _():
        pltpu.sync_copy(mask_hbm, mask_v)
        pltpu.sync_copy(mask_v, msk_sh)

    zf = jnp.zeros((_L,), jnp.float32)

    def zrow_body(i, c):
        for k in range(_D // _L):
            rb0[i, pl.ds(k * _L, _L)] = zf
            rb1[i, pl.ds(k * _L, _L)] = zf
        return c
    lax.fori_loop(0, _GF, zrow_body, 0)
    plsc.subcore_barrier()

    @pl.when(sid != 0)
    def _():
        pltpu.sync_copy(msk_sh, mask_v)

    rbase = sid * _RPTH
    nfull = _RPTH // _GF
    rem = _RPTH - nfull * _GF
    for k in range(nfull):
        pltpu.sync_copy(rb0, acc_sh.at[pl.ds(rbase + k * _GF, _GF)])
    pltpu.sync_copy(rb0.at[pl.ds(0, rem)],
                    acc_sh.at[pl.ds(rbase + nfull * _GF, rem)])

    lo = cid * _HALF

    def ob_body(ob, off):
        pltpu.sync_copy(src_hbm.at[pl.ds(base + ob * _EB, _EB)], src_v)
        pltpu.sync_copy(dst_hbm.at[pl.ds(base + ob * _EB, _EB)], dst_v)

        def cb(i, off):
            s = src_v[pl.ds(i * _L, _L)]
            t = dst_v[pl.ds(i * _L, _L)]
            tl = t - lo
            mv = plsc.load_gather(mask_v, [t])
            m = (mv > 0) & (tl >= 0) & (tl < _HALF)
            plsc.store_compressed(csrc_v.at[pl.ds(off, _L)], s, mask=m)
            plsc.store_compressed(cdst_v.at[pl.ds(off, _L)], tl, mask=m)
            cnt = plsc.all_reduce_population_count(m)
            return off + cnt[0]
        return lax.fori_loop(0, _EB // _L, cb, off)
    kact = lax.fori_loop(0, _EPT // _EB, ob_body, jnp.int32(0))

    spread = jnp.zeros((_L,), jnp.int32) + (wid % _NJ)
    junk = jnp.zeros((_L,), jnp.int32) + (_HALF + (wid % _NJ))
    nbf = (kact + _GF - 1) // _GF
    nb2 = (nbf + 1) // 2
    end = nb2 * 2 * _GF
    iota = lax.iota(jnp.int32, _L)

    def pfb(f, c):
        pos = kact + f * _L + iota
        mfill = pos < end
        plsc.store_scatter(csrc_v, [pos], spread, mask=mfill)
        plsc.store_scatter(cdst_v, [pos], junk, mask=mfill)
        return c
    lax.fori_loop(0, (2 * _GF) // _L + 1, pfb, 0)
    plsc.subcore_barrier()

    def fb(jj, c):
        j0 = jj * 2
        j1 = j0 + 1
        for k in range(_GF // _L):
            ss0[pl.ds(k * _L, _L)] = csrc_v[pl.ds(j0 * _GF + k * _L, _L)]
            sd0[pl.ds(k * _L, _L)] = cdst_v[pl.ds(j0 * _GF + k * _L, _L)]
        cp0 = pltpu.async_copy(rows_hbm.at[ss0], rb0, sem0)
        for k in range(_GF // _L):
            ss1[pl.ds(k * _L, _L)] = csrc_v[pl.ds(j1 * _GF + k * _L, _L)]
            sd1[pl.ds(k * _L, _L)] = cdst_v[pl.ds(j1 * _GF + k * _L, _L)]
        cp1 = pltpu.async_copy(rows_hbm.at[ss1], rb1, sem1)
        cp0.wait()
        pltpu.sync_copy(rb0, acc_sh.at[sd0], add=True)
        cp1.wait()
        pltpu.sync_copy(rb1, acc_sh.at[sd1], add=True)
        return c
    lax.fori_loop(0, nb2, fb, 0)
    plsc.subcore_barrier()

    for k in range(nfull):
        pltpu.sync_copy(acc_sh.at[pl.ds(rbase + k * _GF, _GF)], rb0)
        pltpu.sync_copy(rb0, acc_out.at[cid, pl.ds(rbase + k * _GF, _GF)])
    pltpu.sync_copy(acc_sh.at[pl.ds(rbase + nfull * _GF, rem)],
                    rb1.at[pl.ds(0, rem)])
    pltpu.sync_copy(rb1.at[pl.ds(0, rem)],
                    acc_out.at[cid, pl.ds(rbase + nfull * _GF, rem)])


_sc_scatter = pl.kernel(
    _sc_scatter_body,
    out_type=jax.ShapeDtypeStruct((_NC, _NAH, _D), jnp.float32),
    mesh=_mesh,
    compiler_params=pltpu.CompilerParams(needs_layout_passes=False),
    scratch_types=(pltpu.VMEM((_EB,), jnp.int32),
                   pltpu.VMEM((_EB,), jnp.int32),
                   pltpu.VMEM((_NP,), jnp.int32),
                   pltpu.VMEM((_CAP,), jnp.int32),
                   pltpu.VMEM((_CAP,), jnp.int32),
                   pltpu.VMEM((_GF,), jnp.int32),
                   pltpu.VMEM((_GF,), jnp.int32),
                   pltpu.VMEM((_GF,), jnp.int32),
                   pltpu.VMEM((_GF,), jnp.int32),
                   pltpu.VMEM((_GF, _D), jnp.float32),
                   pltpu.VMEM((_GF, _D), jnp.float32),
                   pltpu.VMEM_SHARED((_NP,), jnp.int32),
                   pltpu.VMEM_SHARED((_NAH, _D), jnp.float32),
                   pltpu.SemaphoreType.DMA,
                   pltpu.SemaphoreType.DMA),
)


def _tc_reduce1_body(degp_ref, m1p_ref, dinv_ref, m1_ref):
    degs = jnp.sum(degp_ref[...], axis=0, keepdims=True) + 1.0
    dinv_ref[...] = lax.rsqrt(degs)
    m1 = jnp.sum(m1p_ref[...], axis=0, keepdims=True) > 0
    col = lax.broadcasted_iota(jnp.int32, (1, _NP), 1)
    m1_ref[...] = jnp.where((col == 0) | m1, 1, 0).astype(jnp.int32)


_tc_reduce1 = pl.pallas_call(
    _tc_reduce1_body,
    out_shape=(jax.ShapeDtypeStruct((1, _NP), jnp.float32),
               jax.ShapeDtypeStruct((1, _NP), jnp.int32)),
)


def _tc_mask2_body(m2p_ref, m1_ref, m2_ref):
    m2 = jnp.sum(m2p_ref[...], axis=0, keepdims=True) > 0
    m2_ref[...] = jnp.where(m2 | (m1_ref[...] > 0), 1, 0).astype(jnp.int32)


_tc_mask2 = pl.pallas_call(
    _tc_mask2_body,
    out_shape=jax.ShapeDtypeStruct((1, _NP), jnp.int32),
)


def _tc_scalemm_body(h_ref, w_ref, dinv_ref, out_ref):
    out_ref[...] = dinv_ref[...] * jnp.dot(
        h_ref[...], w_ref[...], preferred_element_type=jnp.float32)


_tc_scalemm = pl.pallas_call(
    _tc_scalemm_body,
    grid=(_N // _BR,),
    in_specs=[pl.BlockSpec((_BR, _D), lambda i: (i, 0)),
              pl.BlockSpec((_D, _D), lambda i: (0, 0)),
              pl.BlockSpec((_BR, 1), lambda i: (i, 0))],
    out_specs=pl.BlockSpec((_BR, _D), lambda i: (i, 0)),
    out_shape=jax.ShapeDtypeStruct((_N, _D), jnp.float32),
)


def _tc_post_body(acc_ref, hs_ref, dinv_ref, b_ref, g_ref,
                  bn_ref, hprev_ref, out_ref):
    pre = dinv_ref[...] * (acc_ref[...] + hs_ref[...]) + b_ref[...]
    mu = jnp.mean(pre, axis=1, keepdims=True)
    var = jnp.mean((pre - mu) ** 2, axis=1, keepdims=True)
    ln = (pre - mu) * lax.rsqrt(var + 1e-5) * g_ref[...] + bn_ref[...]
    out_ref[...] = jnp.maximum(ln, 0.0) + hprev_ref[...]


_tc_post = pl.pallas_call(
    _tc_post_body,
    grid=(_N // _BR,),
    in_specs=[pl.BlockSpec((_BR, _D), lambda i: (i, 0)),
              pl.BlockSpec((_BR, _D), lambda i: (i, 0)),
              pl.BlockSpec((_BR, 1), lambda i: (i, 0)),
              pl.BlockSpec((1, _D), lambda i: (0, 0)),
              pl.BlockSpec((1, _D), lambda i: (0, 0)),
              pl.BlockSpec((1, _D), lambda i: (0, 0)),
              pl.BlockSpec((_BR, _D), lambda i: (i, 0))],
    out_specs=pl.BlockSpec((_BR, _D), lambda i: (i, 0)),
    out_shape=jax.ShapeDtypeStruct((_N, _D), jnp.float32),
)


def _tc_heads_body(acc0, hs3r, h2r, dinv0, b3, g3, bn3,
                   cvw1, cvb1, cvw2, cvb2, caw1, cab1, caw2, cab2,
                   nvw1, nvb1, nvw2, nvb2, naw1, nab1, naw2, nab2,
                   clsq_ref, navq_ref):
    pre = dinv0[...] * (acc0[...] + hs3r[...]) + b3[...]
    mu = jnp.mean(pre, axis=1, keepdims=True)
    var = jnp.mean((pre - mu) ** 2, axis=1, keepdims=True)
    ln = (pre - mu) * lax.rsqrt(var + 1e-5) * g3[...] + bn3[...]
    cur = jnp.maximum(ln, 0.0) + h2r[...]

    def mlp(w1, b1, w2, b2):
        hmid = jnp.maximum(
            jnp.dot(cur, w1[...], preferred_element_type=jnp.float32)
            + b1[...], 0.0)
        return jnp.dot(hmid, w2[...],
                       preferred_element_type=jnp.float32) + b2[...]

    cv = mlp(cvw1, cvb1, cvw2, cvb2)
    ca = mlp(caw1, cab1, caw2, cab2)
    clsq_ref[...] = cv + ca - jnp.mean(ca, axis=1, keepdims=True)
    nv = mlp(nvw1, nvb1, nvw2, nvb2)
    na = mlp(naw1, nab1, naw2, nab2)
    navq_ref[...] = nv + na - jnp.mean(na, axis=1, keepdims=True)


_tc_heads = pl.pallas_call(
    _tc_heads_body,
    out_shape=(jax.ShapeDtypeStruct((1, 10), jnp.float32),
               jax.ShapeDtypeStruct((1, 32), jnp.float32)),
)


def _reasm(acc):
    return jnp.concatenate([acc[0, :_HALF], acc[1, :_HALF]], axis=0)[:_N]


def kernel(x, edge_index, params):
    p = params
    ei = edge_index.astype(jnp.int32)
    src = ei[0]
    dst = ei[1]

    degp, m1p = _sc_deg_mask1(src, dst)
    dinv2d, mask1_2d = _tc_reduce1(degp, m1p)
    mask1 = mask1_2d.reshape(_NP)
    m2p = _sc_mask2(src, dst, mask1)
    mask2 = _tc_mask2(m2p, mask1_2d).reshape(_NP)
    dinv_col = dinv2d[0, :_N].reshape(_N, 1)

    hs1 = _tc_scalemm(x, p['W1'], dinv_col)
    acc1 = _sc_scatter(src, dst, mask2, hs1)
    h1 = _tc_post(_reasm(acc1), hs1, dinv_col,
                  p['b1'].reshape(1, _D), p['ln1_g'].reshape(1, _D),
                  p['ln1_b'].reshape(1, _D), x)

    hs2 = _tc_scalemm(h1, p['W2'], dinv_col)
    acc2 = _sc_scatter(src, dst, mask1, hs2)
    h2 = _tc_post(_reasm(acc2), hs2, dinv_col,
                  p['b2'].reshape(1, _D), p['ln2_g'].reshape(1, _D),
                  p['ln2_b'].reshape(1, _D), h1)

    hs3 = _tc_scalemm(h2, p['W3'], dinv_col)
    mask0 = jnp.zeros((_NP,), jnp.int32).at[0].set(1)
    acc3 = _sc_scatter(src, dst, mask0, hs3)

    cls_q, nav_q = _tc_heads(
        acc3[0, :1, :], hs3[:1], h2[:1], dinv2d[:, :1],
        p['b3'].reshape(1, _D), p['ln3_g'].reshape(1, _D),
        p['ln3_b'].reshape(1, _D),
        p['cv_W1'], p['cv_b1'].reshape(1, _HID), p['cv_W2'],
        p['cv_b2'].reshape(1, 1),
        p['ca_W1'], p['ca_b1'].reshape(1, _HID), p['ca_W2'],
        p['ca_b2'].reshape(1, 10),
        p['nv_W1'], p['nv_b1'].reshape(1, _HID), p['nv_W2'],
        p['nv_b2'].reshape(1, 1),
        p['na_W1'], p['na_b1'].reshape(1, _HID), p['na_W2'],
        p['na_b2'].reshape(1, 32))
    return (cls_q, nav_q)


# speedup vs baseline: 45.3688x; 1.2005x over previous
import jax
import jax.numpy as jnp
from jax import lax
from jax.experimental import pallas as pl
from jax.experimental.pallas import tpu as pltpu
from jax.experimental.pallas import tpu_sc as plsc

_N = 10000
_E = 320000
_D = 128
_HID = 256
_NP = 10240
_NC = 2
_NS = 16
_NW = _NC * _NS
_EPW = _E // _NW
_L = 16
_G = 128
_NJ = 16
_NA = _NP
_RPT = _NA // _NS
_BR = 2000
_HALF = _NP // _NC
_NAH = _HALF + 512
_RPTH = _NAH // _NS
_EPT = _E // _NS
_EB = 2000
_NBLK = (_EPT + 2 * _G - 1) // _G
_CAP = _NBLK * _G
_GF = 64

_mesh = plsc.VectorSubcoreMesh(core_axis_name="c", subcore_axis_name="s")


def _sc_deg_mask1_body(src_hbm, dst_hbm, deg_out, m1_out,
                       src_v, dst_v, deg_v, m1_v):
    wid = lax.axis_index("s") * _NC + lax.axis_index("c")
    base = wid * _EPW
    pltpu.sync_copy(src_hbm.at[pl.ds(base, _EPW)], src_v)
    pltpu.sync_copy(dst_hbm.at[pl.ds(base, _EPW)], dst_v)
    zi = jnp.zeros((_L,), jnp.int32)
    zf = jnp.zeros((_L,), jnp.float32)

    @plsc.parallel_loop(0, _NP, _L, unroll=4)
    def zero_body(i):
        deg_v[pl.ds(i, _L)] = zf
        m1_v[pl.ds(i, _L)] = zi

    ones = jnp.ones((_L,), jnp.int32)
    onesf = jnp.ones((_L,), jnp.float32)

    @plsc.parallel_loop(0, _EPW, _L, unroll=4)
    def body(i):
        s = src_v[pl.ds(i, _L)]
        t = dst_v[pl.ds(i, _L)]
        plsc.addupdate_scatter(deg_v, [t], onesf)
        plsc.store_scatter(m1_v, [s], ones, mask=t == 0)
    pltpu.sync_copy(deg_v, deg_out.at[wid])
    pltpu.sync_copy(m1_v, m1_out.at[wid])


_sc_deg_mask1 = pl.kernel(
    _sc_deg_mask1_body,
    out_type=(jax.ShapeDtypeStruct((_NW, _NP), jnp.float32),
              jax.ShapeDtypeStruct((_NW, _NP), jnp.int32)),
    mesh=_mesh,
    compiler_params=pltpu.CompilerParams(needs_layout_passes=False),
    scratch_types=(pltpu.VMEM((_EPW,), jnp.int32),
                   pltpu.VMEM((_EPW,), jnp.int32),
                   pltpu.VMEM((_NP,), jnp.float32),
                   pltpu.VMEM((_NP,), jnp.int32)),
)


def _sc_mask2_body(src_hbm, dst_hbm, m1_hbm, m2_out,
                   src_v, dst_v, m1_v, m2_v):
    wid = lax.axis_index("s") * _NC + lax.axis_index("c")
    base = wid * _EPW
    pltpu.sync_copy(src_hbm.at[pl.ds(base, _EPW)], src_v)
    pltpu.sync_copy(dst_hbm.at[pl.ds(base, _EPW)], dst_v)
    pltpu.sync_copy(m1_hbm, m1_v)
    zi = jnp.zeros((_L,), jnp.int32)

    @plsc.parallel_loop(0, _NP, _L, unroll=4)
    def zero_body(i):
        m2_v[pl.ds(i, _L)] = zi

    ones = jnp.ones((_L,), jnp.int32)

    @plsc.parallel_loop(0, _EPW, _L, unroll=4)
    def body(i):
        s = src_v[pl.ds(i, _L)]
        t = dst_v[pl.ds(i, _L)]
        mv = plsc.load_gather(m1_v, [t])
        plsc.store_scatter(m2_v, [s], ones, mask=mv > 0)
    pltpu.sync_copy(m2_v, m2_out.at[wid])


_sc_mask2 = pl.kernel(
    _sc_mask2_body,
    out_type=jax.ShapeDtypeStruct((_NW, _NP), jnp.int32),
    mesh=_mesh,
    compiler_params=pltpu.CompilerParams(needs_layout_passes=False),
    scratch_types=(pltpu.VMEM((_EPW,), jnp.int32),
                   pltpu.VMEM((_EPW,), jnp.int32),
                   pltpu.VMEM((_NP,), jnp.int32),
                   pltpu.VMEM((_NP,), jnp.int32)),
)


def _sc_scatter_body(src_hbm, dst_hbm, mask_hbm, rows_hbm, acc_out,
                     src_v, dst_v, mask_v, csrc_v, cdst_v,
                     ss0, sd0, ss1, sd1, rb0, rb1,
                     msk_sh, acc_sh, sem0, sem1):
    cid = lax.axis_index("c")
    sid = lax.axis_index("s")
    wid = sid * _NC + cid
    base = sid * _EPT

    @pl.when(sid == 0)
    def _():
        pltpu.sync_copy(mask_hbm, mask_v)
        pltpu.sync_copy(mask_v, msk_sh)

    zf = jnp.zeros((_L,), jnp.float32)

    @plsc.parallel_loop(0, _GF, 1, unroll=4)
    def zrow_body(i):
        for k in range(_D // _L):
            rb0[i, pl.ds(k * _L, _L)] = zf
            rb1[i, pl.ds(k * _L, _L)] = zf
    plsc.subcore_barrier()

    @pl.when(sid != 0)
    def _():
        pltpu.sync_copy(msk_sh, mask_v)

    rbase = sid * _RPTH
    nfull = _RPTH // _GF
    rem = _RPTH - nfull * _GF
    for k in range(nfull):
        pltpu.sync_copy(rb0, acc_sh.at[pl.ds(rbase + k * _GF, _GF)])
    pltpu.sync_copy(rb0.at[pl.ds(0, rem)],
                    acc_sh.at[pl.ds(rbase + nfull * _GF, rem)])

    lo = cid * _HALF

    def ob_body(ob, off):
        pltpu.sync_copy(src_hbm.at[pl.ds(base + ob * _EB, _EB)], src_v)
        pltpu.sync_copy(dst_hbm.at[pl.ds(base + ob * _EB, _EB)], dst_v)

        @plsc.parallel_loop(0, _EB, _L, unroll=4, carry=off)
        def cb(i, off):
            s = src_v[pl.ds(i, _L)]
            t = dst_v[pl.ds(i, _L)]
            tl = t - lo
            mv = plsc.load_gather(mask_v, [t])
            m = (mv > 0) & (tl >= 0) & (tl < _HALF)
            plsc.store_compressed(csrc_v.at[pl.ds(off, _L)], s, mask=m)
            plsc.store_compressed(cdst_v.at[pl.ds(off, _L)], tl, mask=m)
            cnt = plsc.all_reduce_population_count(m)
            return off + cnt[0]
        return cb
    kact = lax.fori_loop(0, _EPT // _EB, ob_body, jnp.int32(0))

    spread = jnp.zeros((_L,), jnp.int32) + (wid % _NJ)
    junk = jnp.zeros((_L,), jnp.int32) + (_HALF + (wid % _NJ))
    nbf = (kact + _GF - 1) // _GF
    nb2 = (nbf + 1) // 2
    end = nb2 * 2 * _GF
    iota = lax.iota(jnp.int32, _L)

    def pfb(f, c):
        pos = kact + f * _L + iota
        mfill = pos < end
        plsc.store_scatter(csrc_v, [pos], spread, mask=mfill)
        plsc.store_scatter(cdst_v, [pos], junk, mask=mfill)
        return c
    lax.fori_loop(0, (2 * _GF) // _L + 1, pfb, 0)
    plsc.subcore_barrier()

    def fb(jj, c):
        j0 = jj * 2
        j1 = j0 + 1
        for k in range(_GF // _L):
            ss0[pl.ds(k * _L, _L)] = csrc_v[pl.ds(j0 * _GF + k * _L, _L)]
            sd0[pl.ds(k * _L, _L)] = cdst_v[pl.ds(j0 * _GF + k * _L, _L)]
        cp0 = pltpu.async_copy(rows_hbm.at[ss0], rb0, sem0)
        for k in range(_GF // _L):
            ss1[pl.ds(k * _L, _L)] = csrc_v[pl.ds(j1 * _GF + k * _L, _L)]
            sd1[pl.ds(k * _L, _L)] = cdst_v[pl.ds(j1 * _GF + k * _L, _L)]
        cp1 = pltpu.async_copy(rows_hbm.at[ss1], rb1, sem1)
        cp0.wait()
        pltpu.sync_copy(rb0, acc_sh.at[sd0], add=True)
        cp1.wait()
        pltpu.sync_copy(rb1, acc_sh.at[sd1], add=True)
        return c
    lax.fori_loop(0, nb2, fb, 0)
    plsc.subcore_barrier()

    for k in range(nfull):
        pltpu.sync_copy(acc_sh.at[pl.ds(rbase + k * _GF, _GF)], rb0)
        pltpu.sync_copy(rb0, acc_out.at[cid, pl.ds(rbase + k * _GF, _GF)])
    pltpu.sync_copy(acc_sh.at[pl.ds(rbase + nfull * _GF, rem)],
                    rb1.at[pl.ds(0, rem)])
    pltpu.sync_copy(rb1.at[pl.ds(0, rem)],
                    acc_out.at[cid, pl.ds(rbase + nfull * _GF, rem)])


_sc_scatter = pl.kernel(
    _sc_scatter_body,
    out_type=jax.ShapeDtypeStruct((_NC, _NAH, _D), jnp.float32),
    mesh=_mesh,
    compiler_params=pltpu.CompilerParams(needs_layout_passes=False),
    scratch_types=(pltpu.VMEM((_EB,), jnp.int32),
                   pltpu.VMEM((_EB,), jnp.int32),
                   pltpu.VMEM((_NP,), jnp.int32),
                   pltpu.VMEM((_CAP,), jnp.int32),
                   pltpu.VMEM((_CAP,), jnp.int32),
                   pltpu.VMEM((_GF,), jnp.int32),
                   pltpu.VMEM((_GF,), jnp.int32),
                   pltpu.VMEM((_GF,), jnp.int32),
                   pltpu.VMEM((_GF,), jnp.int32),
                   pltpu.VMEM((_GF, _D), jnp.float32),
                   pltpu.VMEM((_GF, _D), jnp.float32),
                   pltpu.VMEM_SHARED((_NP,), jnp.int32),
                   pltpu.VMEM_SHARED((_NAH, _D), jnp.float32),
                   pltpu.SemaphoreType.DMA,
                   pltpu.SemaphoreType.DMA),
)


def _tc_reduce1_body(degp_ref, m1p_ref, dinv_ref, m1_ref):
    degs = jnp.sum(degp_ref[...], axis=0, keepdims=True) + 1.0
    dinv_ref[...] = lax.rsqrt(degs)
    m1 = jnp.sum(m1p_ref[...], axis=0, keepdims=True) > 0
    col = lax.broadcasted_iota(jnp.int32, (1, _NP), 1)
    m1_ref[...] = jnp.where((col == 0) | m1, 1, 0).astype(jnp.int32)


_tc_reduce1 = pl.pallas_call(
    _tc_reduce1_body,
    out_shape=(jax.ShapeDtypeStruct((1, _NP), jnp.float32),
               jax.ShapeDtypeStruct((1, _NP), jnp.int32)),
)


def _tc_mask2_body(m2p_ref, m1_ref, m2_ref):
    m2 = jnp.sum(m2p_ref[...], axis=0, keepdims=True) > 0
    m2_ref[...] = jnp.where(m2 | (m1_ref[...] > 0), 1, 0).astype(jnp.int32)


_tc_mask2 = pl.pallas_call(
    _tc_mask2_body,
    out_shape=jax.ShapeDtypeStruct((1, _NP), jnp.int32),
)


def _tc_scalemm_body(h_ref, w_ref, dinv_ref, out_ref):
    out_ref[...] = dinv_ref[...] * jnp.dot(
        h_ref[...], w_ref[...], preferred_element_type=jnp.float32)


_tc_scalemm = pl.pallas_call(
    _tc_scalemm_body,
    grid=(_N // _BR,),
    in_specs=[pl.BlockSpec((_BR, _D), lambda i: (i, 0)),
              pl.BlockSpec((_D, _D), lambda i: (0, 0)),
              pl.BlockSpec((_BR, 1), lambda i: (i, 0))],
    out_specs=pl.BlockSpec((_BR, _D), lambda i: (i, 0)),
    out_shape=jax.ShapeDtypeStruct((_N, _D), jnp.float32),
)


def _tc_post_body(acc_ref, hs_ref, dinv_ref, b_ref, g_ref,
                  bn_ref, hprev_ref, out_ref):
    pre = dinv_ref[...] * (acc_ref[...] + hs_ref[...]) + b_ref[...]
    mu = jnp.mean(pre, axis=1, keepdims=True)
    var = jnp.mean((pre - mu) ** 2, axis=1, keepdims=True)
    ln = (pre - mu) * lax.rsqrt(var + 1e-5) * g_ref[...] + bn_ref[...]
    out_ref[...] = jnp.maximum(ln, 0.0) + hprev_ref[...]


_tc_post = pl.pallas_call(
    _tc_post_body,
    grid=(_N // _BR,),
    in_specs=[pl.BlockSpec((_BR, _D), lambda i: (i, 0)),
              pl.BlockSpec((_BR, _D), lambda i: (i, 0)),
              pl.BlockSpec((_BR, 1), lambda i: (i, 0)),
              pl.BlockSpec((1, _D), lambda i: (0, 0)),
              pl.BlockSpec((1, _D), lambda i: (0, 0)),
              pl.BlockSpec((1, _D), lambda i: (0, 0)),
              pl.BlockSpec((_BR, _D), lambda i: (i, 0))],
    out_specs=pl.BlockSpec((_BR, _D), lambda i: (i, 0)),
    out_shape=jax.ShapeDtypeStruct((_N, _D), jnp.float32),
)


def _tc_heads_body(acc0, hs3r, h2r, dinv0, b3, g3, bn3,
                   cvw1, cvb1, cvw2, cvb2, caw1, cab1, caw2, cab2,
                   nvw1, nvb1, nvw2, nvb2, naw1, nab1, naw2, nab2,
                   clsq_ref, navq_ref):
    pre = dinv0[...] * (acc0[...] + hs3r[...]) + b3[...]
    mu = jnp.mean(pre, axis=1, keepdims=True)
    var = jnp.mean((pre - mu) ** 2, axis=1, keepdims=True)
    ln = (pre - mu) * lax.rsqrt(var + 1e-5) * g3[...] + bn3[...]
    cur = jnp.maximum(ln, 0.0) + h2r[...]

    def mlp(w1, b1, w2, b2):
        hmid = jnp.maximum(
            jnp.dot(cur, w1[...], preferred_element_type=jnp.float32)
            + b1[...], 0.0)
        return jnp.dot(hmid, w2[...],
                       preferred_element_type=jnp.float32) + b2[...]

    cv = mlp(cvw1, cvb1, cvw2, cvb2)
    ca = mlp(caw1, cab1, caw2, cab2)
    clsq_ref[...] = cv + ca - jnp.mean(ca, axis=1, keepdims=True)
    nv = mlp(nvw1, nvb1, nvw2, nvb2)
    na = mlp(naw1, nab1, naw2, nab2)
    navq_ref[...] = nv + na - jnp.mean(na, axis=1, keepdims=True)


_tc_heads = pl.pallas_call(
    _tc_heads_body,
    out_shape=(jax.ShapeDtypeStruct((1, 10), jnp.float32),
               jax.ShapeDtypeStruct((1, 32), jnp.float32)),
)


def _reasm(acc):
    return jnp.concatenate([acc[0, :_HALF], acc[1, :_HALF]], axis=0)[:_N]


def kernel(x, edge_index, params):
    p = params
    ei = edge_index.astype(jnp.int32)
    src = ei[0]
    dst = ei[1]

    degp, m1p = _sc_deg_mask1(src, dst)
    dinv2d, mask1_2d = _tc_reduce1(degp, m1p)
    mask1 = mask1_2d.reshape(_NP)
    m2p = _sc_mask2(src, dst, mask1)
    mask2 = _tc_mask2(m2p, mask1_2d).reshape(_NP)
    dinv_col = dinv2d[0, :_N].reshape(_N, 1)

    hs1 = _tc_scalemm(x, p['W1'], dinv_col)
    acc1 = _sc_scatter(src, dst, mask2, hs1)
    h1 = _tc_post(_reasm(acc1), hs1, dinv_col,
                  p['b1'].reshape(1, _D), p['ln1_g'].reshape(1, _D),
                  p['ln1_b'].reshape(1, _D), x)

    hs2 = _tc_scalemm(h1, p['W2'], dinv_col)
    acc2 = _sc_scatter(src, dst, mask1, hs2)
    h2 = _tc_post(_reasm(acc2), hs2, dinv_col,
                  p['b2'].reshape(1, _D), p['ln2_g'].reshape(1, _D),
                  p['ln2_b'].reshape(1, _D), h1)

    hs3 = _tc_scalemm(h2, p['W3'], dinv_col)
    mask0 = jnp.zeros((_NP,), jnp.int32).at[0].set(1)
    acc3 = _sc_scatter(src, dst, mask0, hs3)

    cls_q, nav_q = _tc_heads(
        acc3[0, :1, :], hs3[:1], h2[:1], dinv2d[:, :1],
        p['b3'].reshape(1, _D), p['ln3_g'].reshape(1, _D),
        p['ln3_b'].reshape(1, _D),
        p['cv_W1'], p['cv_b1'].reshape(1, _HID), p['cv_W2'],
        p['cv_b2'].reshape(1, 1),
        p['ca_W1'], p['ca_b1'].reshape(1, _HID), p['ca_W2'],
        p['ca_b2'].reshape(1, 10),
        p['nv_W1'], p['nv_b1'].reshape(1, _HID), p['nv_W2'],
        p['nv_b2'].reshape(1, 1),
        p['na_W1'], p['na_b1'].reshape(1, _HID), p['na_W2'],
        p['na_b2'].reshape(1, 32))
    return (cls_q, nav_q)


# speedup vs baseline: 50.7089x; 1.1177x over previous
import jax
import jax.numpy as jnp
from jax import lax
from jax.experimental import pallas as pl
from jax.experimental.pallas import tpu as pltpu
from jax.experimental.pallas import tpu_sc as plsc

_N = 10000
_E = 320000
_D = 128
_HID = 256
_NP = 10240
_NC = 2
_NS = 16
_NW = _NC * _NS
_EPW = _E // _NW
_L = 16
_G = 128
_NJ = 16
_NA = _NP
_RPT = _NA // _NS
_BR = 1280
_HALF = _NP // _NC
_NAH = _HALF + 512
_RPTH = _NAH // _NS
_EPT = _E // _NS
_EB = 2000
_NBLK = (_EPT + 2 * _G - 1) // _G
_CAP = _NBLK * _G
_GF = 64

_mesh = plsc.VectorSubcoreMesh(core_axis_name="c", subcore_axis_name="s")


def _sc_deg_mask1_body(src_hbm, dst_hbm, deg_out, m1_out,
                       src_v, dst_v, deg_v, m1_v):
    wid = lax.axis_index("s") * _NC + lax.axis_index("c")
    base = wid * _EPW
    pltpu.sync_copy(src_hbm.at[pl.ds(base, _EPW)], src_v)
    pltpu.sync_copy(dst_hbm.at[pl.ds(base, _EPW)], dst_v)
    zi = jnp.zeros((_L,), jnp.int32)
    zf = jnp.zeros((_L,), jnp.float32)

    @plsc.parallel_loop(0, _NP, _L, unroll=4)
    def zero_body(i):
        deg_v[pl.ds(i, _L)] = zf
        m1_v[pl.ds(i, _L)] = zi

    ones = jnp.ones((_L,), jnp.int32)
    onesf = jnp.ones((_L,), jnp.float32)

    @plsc.parallel_loop(0, _EPW, _L, unroll=4)
    def body(i):
        s = src_v[pl.ds(i, _L)]
        t = dst_v[pl.ds(i, _L)]
        plsc.addupdate_scatter(deg_v, [t], onesf)
        plsc.store_scatter(m1_v, [s], ones, mask=t == 0)
    pltpu.sync_copy(deg_v, deg_out.at[wid])
    pltpu.sync_copy(m1_v, m1_out.at[wid])


_sc_deg_mask1 = pl.kernel(
    _sc_deg_mask1_body,
    out_type=(jax.ShapeDtypeStruct((_NW, _NP), jnp.float32),
              jax.ShapeDtypeStruct((_NW, _NP), jnp.int32)),
    mesh=_mesh,
    compiler_params=pltpu.CompilerParams(needs_layout_passes=False),
    scratch_types=(pltpu.VMEM((_EPW,), jnp.int32),
                   pltpu.VMEM((_EPW,), jnp.int32),
                   pltpu.VMEM((_NP,), jnp.float32),
                   pltpu.VMEM((_NP,), jnp.int32)),
)


def _sc_mask2_body(src_hbm, dst_hbm, m1_hbm, m2_out,
                   src_v, dst_v, m1_v, m2_v):
    wid = lax.axis_index("s") * _NC + lax.axis_index("c")
    base = wid * _EPW
    pltpu.sync_copy(src_hbm.at[pl.ds(base, _EPW)], src_v)
    pltpu.sync_copy(dst_hbm.at[pl.ds(base, _EPW)], dst_v)
    pltpu.sync_copy(m1_hbm, m1_v)
    zi = jnp.zeros((_L,), jnp.int32)

    @plsc.parallel_loop(0, _NP, _L, unroll=4)
    def zero_body(i):
        m2_v[pl.ds(i, _L)] = zi

    ones = jnp.ones((_L,), jnp.int32)

    @plsc.parallel_loop(0, _EPW, _L, unroll=4)
    def body(i):
        s = src_v[pl.ds(i, _L)]
        t = dst_v[pl.ds(i, _L)]
        mv = plsc.load_gather(m1_v, [t])
        plsc.store_scatter(m2_v, [s], ones, mask=mv > 0)
    pltpu.sync_copy(m2_v, m2_out.at[wid])


_sc_mask2 = pl.kernel(
    _sc_mask2_body,
    out_type=jax.ShapeDtypeStruct((_NW, _NP), jnp.int32),
    mesh=_mesh,
    compiler_params=pltpu.CompilerParams(needs_layout_passes=False),
    scratch_types=(pltpu.VMEM((_EPW,), jnp.int32),
                   pltpu.VMEM((_EPW,), jnp.int32),
                   pltpu.VMEM((_NP,), jnp.int32),
                   pltpu.VMEM((_NP,), jnp.int32)),
)


def _sc_scatter_body(src_hbm, dst_hbm, mask_hbm, rows_hbm, acc_out,
                     src_v, dst_v, mask_v, csrc_v, cdst_v,
                     ss0, sd0, ss1, sd1, rb0, rb1,
                     msk_sh, acc_sh, sem0, sem1):
    cid = lax.axis_index("c")
    sid = lax.axis_index("s")
    wid = sid * _NC + cid
    base = sid * _EPT

    @pl.when(sid == 0)
    def _():
        pltpu.sync_copy(mask_hbm, mask_v)
        pltpu.sync_copy(mask_v, msk_sh)

    zf = jnp.zeros((_L,), jnp.float32)

    @plsc.parallel_loop(0, _GF, 1, unroll=4)
    def zrow_body(i):
        for k in range(_D // _L):
            rb0[i, pl.ds(k * _L, _L)] = zf
            rb1[i, pl.ds(k * _L, _L)] = zf
    plsc.subcore_barrier()

    @pl.when(sid != 0)
    def _():
        pltpu.sync_copy(msk_sh, mask_v)

    rbase = sid * _RPTH
    nfull = _RPTH // _GF
    rem = _RPTH - nfull * _GF
    for k in range(nfull):
        pltpu.sync_copy(rb0, acc_sh.at[pl.ds(rbase + k * _GF, _GF)])
    pltpu.sync_copy(rb0.at[pl.ds(0, rem)],
                    acc_sh.at[pl.ds(rbase + nfull * _GF, rem)])

    lo = cid * _HALF

    def ob_body(ob, off):
        pltpu.sync_copy(src_hbm.at[pl.ds(base + ob * _EB, _EB)], src_v)
        pltpu.sync_copy(dst_hbm.at[pl.ds(base + ob * _EB, _EB)], dst_v)

        @plsc.parallel_loop(0, _EB, _L, unroll=4, carry=off)
        def cb(i, off):
            s = src_v[pl.ds(i, _L)]
            t = dst_v[pl.ds(i, _L)]
            tl = t - lo
            mv = plsc.load_gather(mask_v, [t])
            m = (mv > 0) & (tl >= 0) & (tl < _HALF)
            plsc.store_compressed(csrc_v.at[pl.ds(off, _L)], s, mask=m)
            plsc.store_compressed(cdst_v.at[pl.ds(off, _L)], tl, mask=m)
            cnt = plsc.all_reduce_population_count(m)
            return off + cnt[0]
        return cb
    kact = lax.fori_loop(0, _EPT // _EB, ob_body, jnp.int32(0))

    spread = jnp.zeros((_L,), jnp.int32) + (wid % _NJ)
    junk = jnp.zeros((_L,), jnp.int32) + (_HALF + (wid % _NJ))
    nbf = (kact + _GF - 1) // _GF
    nb2 = (nbf + 1) // 2
    end = nb2 * 2 * _GF
    iota = lax.iota(jnp.int32, _L)

    def pfb(f, c):
        pos = kact + f * _L + iota
        mfill = pos < end
        plsc.store_scatter(csrc_v, [pos], spread, mask=mfill)
        plsc.store_scatter(cdst_v, [pos], junk, mask=mfill)
        return c
    lax.fori_loop(0, (2 * _GF) // _L + 1, pfb, 0)
    plsc.subcore_barrier()

    def fb(jj, c):
        j0 = jj * 2
        j1 = j0 + 1
        for k in range(_GF // _L):
            ss0[pl.ds(k * _L, _L)] = csrc_v[pl.ds(j0 * _GF + k * _L, _L)]
            sd0[pl.ds(k * _L, _L)] = cdst_v[pl.ds(j0 * _GF + k * _L, _L)]
        cp0 = pltpu.async_copy(rows_hbm.at[ss0], rb0, sem0)
        for k in range(_GF // _L):
            ss1[pl.ds(k * _L, _L)] = csrc_v[pl.ds(j1 * _GF + k * _L, _L)]
            sd1[pl.ds(k * _L, _L)] = cdst_v[pl.ds(j1 * _GF + k * _L, _L)]
        cp1 = pltpu.async_copy(rows_hbm.at[ss1], rb1, sem1)
        cp0.wait()
        pltpu.sync_copy(rb0, acc_sh.at[sd0], add=True)
        cp1.wait()
        pltpu.sync_copy(rb1, acc_sh.at[sd1], add=True)
        return c
    lax.fori_loop(0, nb2, fb, 0)
    plsc.subcore_barrier()

    for k in range(nfull):
        pltpu.sync_copy(acc_sh.at[pl.ds(rbase + k * _GF, _GF)], rb0)
        pltpu.sync_copy(rb0, acc_out.at[cid, pl.ds(rbase + k * _GF, _GF)])
    pltpu.sync_copy(acc_sh.at[pl.ds(rbase + nfull * _GF, rem)],
                    rb1.at[pl.ds(0, rem)])
    pltpu.sync_copy(rb1.at[pl.ds(0, rem)],
                    acc_out.at[cid, pl.ds(rbase + nfull * _GF, rem)])


_sc_scatter = pl.kernel(
    _sc_scatter_body,
    out_type=jax.ShapeDtypeStruct((_NC, _NAH, _D), jnp.float32),
    mesh=_mesh,
    compiler_params=pltpu.CompilerParams(needs_layout_passes=False),
    scratch_types=(pltpu.VMEM((_EB,), jnp.int32),
                   pltpu.VMEM((_EB,), jnp.int32),
                   pltpu.VMEM((_NP,), jnp.int32),
                   pltpu.VMEM((_CAP,), jnp.int32),
                   pltpu.VMEM((_CAP,), jnp.int32),
                   pltpu.VMEM((_GF,), jnp.int32),
                   pltpu.VMEM((_GF,), jnp.int32),
                   pltpu.VMEM((_GF,), jnp.int32),
                   pltpu.VMEM((_GF,), jnp.int32),
                   pltpu.VMEM((_GF, _D), jnp.float32),
                   pltpu.VMEM((_GF, _D), jnp.float32),
                   pltpu.VMEM_SHARED((_NP,), jnp.int32),
                   pltpu.VMEM_SHARED((_NAH, _D), jnp.float32),
                   pltpu.SemaphoreType.DMA,
                   pltpu.SemaphoreType.DMA),
)


def _tc_reduce1_body(degp_ref, m1p_ref, dinv_ref, m1_ref):
    degs = jnp.sum(degp_ref[...], axis=0, keepdims=True) + 1.0
    dinv_ref[...] = lax.rsqrt(degs)
    m1 = jnp.sum(m1p_ref[...], axis=0, keepdims=True) > 0
    col = lax.broadcasted_iota(jnp.int32, (1, _NP), 1)
    m1_ref[...] = jnp.where((col == 0) | (m1 & (col < _N)), 1, 0)


_tc_reduce1 = pl.pallas_call(
    _tc_reduce1_body,
    out_shape=(jax.ShapeDtypeStruct((1, _NP), jnp.float32),
               jax.ShapeDtypeStruct((1, _NP), jnp.int32)),
)


def _tc_mask2_body(m2p_ref, m1_ref, m2_ref):
    m2 = jnp.sum(m2p_ref[...], axis=0, keepdims=True) > 0
    col = lax.broadcasted_iota(jnp.int32, (1, _NP), 1)
    m2_ref[...] = jnp.where((m2 | (m1_ref[...] > 0)) & (col < _N), 1, 0)


_tc_mask2 = pl.pallas_call(
    _tc_mask2_body,
    out_shape=jax.ShapeDtypeStruct((1, _NP), jnp.int32),
)


def _tc_scalemm_body(h_ref, w_ref, dinv_ref, out_ref):
    out_ref[...] = dinv_ref[...] * jnp.dot(
        h_ref[...], w_ref[...], preferred_element_type=jnp.float32)


_tc_scalemm = pl.pallas_call(
    _tc_scalemm_body,
    grid=(_NP // _BR,),
    in_specs=[pl.BlockSpec((_BR, _D), lambda i: (i, 0)),
              pl.BlockSpec((_D, _D), lambda i: (0, 0)),
              pl.BlockSpec((_BR, 1), lambda i: (i, 0))],
    out_specs=pl.BlockSpec((_BR, _D), lambda i: (i, 0)),
    out_shape=jax.ShapeDtypeStruct((_NP, _D), jnp.float32),
)


def _tc_postmm_body(acc_ref, hs_ref, dinv_ref, b_ref, g_ref,
                    bn_ref, hprev_ref, w_ref, h_ref, hsn_ref):
    pre = dinv_ref[...] * (acc_ref[0] + hs_ref[...]) + b_ref[...]
    mu = jnp.mean(pre, axis=1, keepdims=True)
    var = jnp.mean((pre - mu) ** 2, axis=1, keepdims=True)
    ln = (pre - mu) * lax.rsqrt(var + 1e-5) * g_ref[...] + bn_ref[...]
    h = jnp.maximum(ln, 0.0) + hprev_ref[...]
    h_ref[...] = h
    hsn_ref[...] = dinv_ref[...] * jnp.dot(
        h, w_ref[...], preferred_element_type=jnp.float32)


_tc_postmm = pl.pallas_call(
    _tc_postmm_body,
    grid=(_NP // _BR,),
    in_specs=[pl.BlockSpec((1, _BR, _D), lambda i: (i // 4, i % 4, 0)),
              pl.BlockSpec((_BR, _D), lambda i: (i, 0)),
              pl.BlockSpec((_BR, 1), lambda i: (i, 0)),
              pl.BlockSpec((1, _D), lambda i: (0, 0)),
              pl.BlockSpec((1, _D), lambda i: (0, 0)),
              pl.BlockSpec((1, _D), lambda i: (0, 0)),
              pl.BlockSpec((_BR, _D), lambda i: (i, 0)),
              pl.BlockSpec((_D, _D), lambda i: (0, 0))],
    out_specs=(pl.BlockSpec((_BR, _D), lambda i: (i, 0)),
               pl.BlockSpec((_BR, _D), lambda i: (i, 0))),
    out_shape=(jax.ShapeDtypeStruct((_NP, _D), jnp.float32),
               jax.ShapeDtypeStruct((_NP, _D), jnp.float32)),
)


def _tc_heads_body(acc0, hs3r, h2r, dinv0, b3, g3, bn3,
                   cvw1, cvb1, cvw2, cvb2, caw1, cab1, caw2, cab2,
                   nvw1, nvb1, nvw2, nvb2, naw1, nab1, naw2, nab2,
                   clsq_ref, navq_ref):
    pre = dinv0[...] * (acc0[...] + hs3r[...]) + b3[...]
    mu = jnp.mean(pre, axis=1, keepdims=True)
    var = jnp.mean((pre - mu) ** 2, axis=1, keepdims=True)
    ln = (pre - mu) * lax.rsqrt(var + 1e-5) * g3[...] + bn3[...]
    cur = jnp.maximum(ln, 0.0) + h2r[...]

    def mlp(w1, b1, w2, b2):
        hmid = jnp.maximum(
            jnp.dot(cur, w1[...], preferred_element_type=jnp.float32)
            + b1[...], 0.0)
        return jnp.dot(hmid, w2[...],
                       preferred_element_type=jnp.float32) + b2[...]

    cv = mlp(cvw1, cvb1, cvw2, cvb2)
    ca = mlp(caw1, cab1, caw2, cab2)
    clsq_ref[...] = cv + ca - jnp.mean(ca, axis=1, keepdims=True)
    nv = mlp(nvw1, nvb1, nvw2, nvb2)
    na = mlp(naw1, nab1, naw2, nab2)
    navq_ref[...] = nv + na - jnp.mean(na, axis=1, keepdims=True)


_tc_heads = pl.pallas_call(
    _tc_heads_body,
    out_shape=(jax.ShapeDtypeStruct((1, 10), jnp.float32),
               jax.ShapeDtypeStruct((1, 32), jnp.float32)),
)


def kernel(x, edge_index, params):
    p = params
    ei = edge_index.astype(jnp.int32)
    src = ei[0]
    dst = ei[1]
    xp = jnp.pad(x, ((0, _NP - _N), (0, 0)))

    degp, m1p = _sc_deg_mask1(src, dst)
    dinv2d, mask1_2d = _tc_reduce1(degp, m1p)
    mask1 = mask1_2d.reshape(_NP)
    m2p = _sc_mask2(src, dst, mask1)
    mask2 = _tc_mask2(m2p, mask1_2d).reshape(_NP)
    dinv_col = dinv2d.reshape(_NP, 1)

    hs1 = _tc_scalemm(xp, p['W1'], dinv_col)
    acc1 = _sc_scatter(src, dst, mask2, hs1)
    h1, hs2 = _tc_postmm(acc1, hs1, dinv_col,
                         p['b1'].reshape(1, _D), p['ln1_g'].reshape(1, _D),
                         p['ln1_b'].reshape(1, _D), xp, p['W2'])
    acc2 = _sc_scatter(src, dst, mask1, hs2)
    h2, hs3 = _tc_postmm(acc2, hs2, dinv_col,
                         p['b2'].reshape(1, _D), p['ln2_g'].reshape(1, _D),
                         p['ln2_b'].reshape(1, _D), h1, p['W3'])
    mask0 = jnp.zeros((_NP,), jnp.int32).at[0].set(1)
    acc3 = _sc_scatter(src, dst, mask0, hs3)

    cls_q, nav_q = _tc_heads(
        acc3[0, :1, :], hs3[:1], h2[:1], dinv2d[:, :1],
        p['b3'].reshape(1, _D), p['ln3_g'].reshape(1, _D),
        p['ln3_b'].reshape(1, _D),
        p['cv_W1'], p['cv_b1'].reshape(1, _HID), p['cv_W2'],
        p['cv_b2'].reshape(1, 1),
        p['ca_W1'], p['ca_b1'].reshape(1, _HID), p['ca_W2'],
        p['ca_b2'].reshape(1, 10),
        p['nv_W1'], p['nv_b1'].reshape(1, _HID), p['nv_W2'],
        p['nv_b2'].reshape(1, 1),
        p['na_W1'], p['na_b1'].reshape(1, _HID), p['na_W2'],
        p['na_b2'].reshape(1, 32))
    return (cls_q, nav_q)
